# baseline (speedup 1.0000x reference)
import functools

import jax
import jax.numpy as jnp
from jax import lax
from jax.experimental import pallas as pl
from jax.experimental.pallas import tpu as pltpu

_B = 256
_DIN = 1024
_D = 2048
_C = 1000
_K = 100
_N = _C + _B
_NP = 1280
_CP = 1024
_RB = 256
_CB = 256


def _feat_body(x_ref, w_ref, b_ref, o_ref):
    o_ref[...] = (
        jax.lax.dot_general(
            x_ref[...], w_ref[...], (((1,), (1,)), ((), ())),
            preferred_element_type=jnp.float32)
        + b_ref[...]
    )


def _featurize(x, feat_W, feat_b):
    fb = feat_b.reshape(1, _D)
    return pl.pallas_call(
        _feat_body,
        grid=(4,),
        in_specs=[
            pl.BlockSpec((_B, _DIN), lambda j: (0, 0)),
            pl.BlockSpec((_D // 4, _DIN), lambda j: (j, 0)),
            pl.BlockSpec((1, _D // 4), lambda j: (0, j)),
        ],
        out_specs=pl.BlockSpec((_B, _D // 4), lambda j: (0, j)),
        out_shape=jax.ShapeDtypeStruct((_B, _D), jnp.float32),
    )(x, feat_W, fb)


def _stats_body(sup_ref, w_ref, b_ref,
                ent_ref, idx_ref, rn2_ref,
                m_sc, s_sc, t_sc, bv_sc, bi_sc):
    j = pl.program_id(1)
    ncol = pl.num_programs(1)
    lhs = sup_ref[...]
    logits = jax.lax.dot_general(
        lhs, w_ref[...], (((1,), (1,)), ((), ())),
        preferred_element_type=jnp.float32) + b_ref[...]

    bm = jnp.max(logits, axis=1, keepdims=True)
    e = jnp.exp(logits - bm)
    bs = jnp.sum(e, axis=1, keepdims=True)
    bt = jnp.sum(e * logits, axis=1, keepdims=True)
    cols = jax.lax.broadcasted_iota(jnp.int32, logits.shape, 1) + j * _CB
    bi = jnp.min(jnp.where(logits == bm, cols, jnp.int32(2**30)),
                 axis=1, keepdims=True)

    @pl.when(j == 0)
    def _init():
        m_sc[...] = bm
        s_sc[...] = bs
        t_sc[...] = bt
        bv_sc[...] = bm
        bi_sc[...] = bi
        rn2_ref[...] = jnp.sum(lhs * lhs, axis=1, keepdims=True)

    @pl.when(j > 0)
    def _update():
        m_old = m_sc[...]
        m_new = jnp.maximum(m_old, bm)
        sc_old = jnp.exp(m_old - m_new)
        sc_blk = jnp.exp(bm - m_new)
        m_sc[...] = m_new
        s_sc[...] = s_sc[...] * sc_old + bs * sc_blk
        t_sc[...] = t_sc[...] * sc_old + bt * sc_blk
        take = bm > bv_sc[...]
        bv_sc[...] = jnp.maximum(bv_sc[...], bm)
        bi_sc[...] = jnp.where(take, bi, bi_sc[...])

    @pl.when(j == ncol - 1)
    def _final():
        s = s_sc[...]
        logz = m_sc[...] + jnp.log(s)
        ent_ref[...] = logz - t_sc[...] / s
        idx_ref[...] = bi_sc[...]


def _row_stats(supports_pad, clsW_pad, clsb_pad):
    grid = (_NP // _RB, _CP // _CB)
    return pl.pallas_call(
        _stats_body,
        grid=grid,
        in_specs=[
            pl.BlockSpec((_RB, _D), lambda i, j: (i, 0)),
            pl.BlockSpec((_CB, _D), lambda i, j: (j, 0)),
            pl.BlockSpec((1, _CB), lambda i, j: (0, j)),
        ],
        out_specs=[
            pl.BlockSpec((_RB, 1), lambda i, j: (i, 0)),
            pl.BlockSpec((_RB, 1), lambda i, j: (i, 0)),
            pl.BlockSpec((_RB, 1), lambda i, j: (i, 0)),
        ],
        out_shape=[
            jax.ShapeDtypeStruct((_NP, 1), jnp.float32),
            jax.ShapeDtypeStruct((_NP, 1), jnp.int32),
            jax.ShapeDtypeStruct((_NP, 1), jnp.float32),
        ],
        scratch_shapes=[
            pltpu.VMEM((_RB, 1), jnp.float32),
            pltpu.VMEM((_RB, 1), jnp.float32),
            pltpu.VMEM((_RB, 1), jnp.float32),
            pltpu.VMEM((_RB, 1), jnp.float32),
            pltpu.VMEM((_RB, 1), jnp.int32),
        ],
        compiler_params=pltpu.CompilerParams(
            dimension_semantics=("parallel", "arbitrary")),
    )(supports_pad, clsW_pad, clsb_pad)


def _rank_body(ent_r_ref, y_r_ref, ent_c_ref, y_c_ref, rn2_ref, coef_ref):
    i = pl.program_id(0)
    ent_r = ent_r_ref[...]
    y_r = y_r_ref[...]
    ent_c = ent_c_ref[...]
    y_c = y_c_ref[...]
    idx_r = jax.lax.broadcasted_iota(jnp.int32, (_RB, _NP), 1)
    idx_c = jax.lax.broadcasted_iota(jnp.int32, (_RB, _NP), 0) + i * _RB
    same = y_r == y_c
    earlier = (ent_r < ent_c) | ((ent_r == ent_c) & (idx_r < idx_c))
    rank = jnp.sum((same & earlier).astype(jnp.float32), axis=1,
                   keepdims=True)
    valid = (idx_c[:, :1] < _N) & (rank < _K)
    coef_ref[...] = jnp.where(
        valid, jax.lax.rsqrt(jnp.maximum(rn2_ref[...], 1e-24)), 0.0)


def _rank_coef(ent, yidx, rn2):
    ent_row = ent.reshape(1, _NP)
    y_row = yidx.reshape(1, _NP)
    return pl.pallas_call(
        _rank_body,
        grid=(_NP // _RB,),
        in_specs=[
            pl.BlockSpec((1, _NP), lambda i: (0, 0)),
            pl.BlockSpec((1, _NP), lambda i: (0, 0)),
            pl.BlockSpec((_RB, 1), lambda i: (i, 0)),
            pl.BlockSpec((_RB, 1), lambda i: (i, 0)),
            pl.BlockSpec((_RB, 1), lambda i: (i, 0)),
        ],
        out_specs=pl.BlockSpec((_RB, 1), lambda i: (i, 0)),
        out_shape=jax.ShapeDtypeStruct((_NP, 1), jnp.float32),
    )(ent_row, y_row, ent, yidx, rn2)


def _scatter_body(y_ref, coef_ref, sup_ref, o_ref, acc_sc):
    c = pl.program_id(0)
    i = pl.program_id(1)
    ni = pl.num_programs(1)
    classes = jax.lax.broadcasted_iota(jnp.int32, (_RB, _CB), 1) + c * _CB
    onehot = jnp.where(y_ref[...] == classes, coef_ref[...], 0.0)
    contrib = jax.lax.dot_general(
        onehot, sup_ref[...], (((0,), (0,)), ((), ())),
        preferred_element_type=jnp.float32)

    @pl.when(i == 0)
    def _init():
        acc_sc[...] = contrib

    @pl.when(i > 0)
    def _acc():
        acc_sc[...] += contrib

    @pl.when(i == ni - 1)
    def _final():
        o_ref[...] = acc_sc[...]


def _class_scatter(yidx, coef, supports_pad):
    return pl.pallas_call(
        _scatter_body,
        grid=(_CP // _CB, _NP // _RB),
        in_specs=[
            pl.BlockSpec((_RB, 1), lambda c, i: (i, 0)),
            pl.BlockSpec((_RB, 1), lambda c, i: (i, 0)),
            pl.BlockSpec((_RB, _D), lambda c, i: (i, 0)),
        ],
        out_specs=pl.BlockSpec((_CB, _D), lambda c, i: (c, 0)),
        out_shape=jax.ShapeDtypeStruct((_CP, _D), jnp.float32),
        scratch_shapes=[pltpu.VMEM((_CB, _D), jnp.float32)],
        compiler_params=pltpu.CompilerParams(
            dimension_semantics=("parallel", "arbitrary")),
    )(yidx, coef, supports_pad)


def _out_body(wt_ref, z_ref, o_ref):
    wt = wt_ref[...]
    rn2 = jnp.sum(wt * wt, axis=1, keepdims=True)
    wn = wt * jax.lax.rsqrt(jnp.maximum(rn2, 1e-24))
    o_ref[...] = jax.lax.dot_general(
        z_ref[...], wn, (((1,), (1,)), ((), ())),
        preferred_element_type=jnp.float32)


def _final_out(weightsT, z):
    return pl.pallas_call(
        _out_body,
        grid=(_CP // _CB,),
        in_specs=[
            pl.BlockSpec((_CB, _D), lambda j: (j, 0)),
            pl.BlockSpec((_B, _D), lambda j: (0, 0)),
        ],
        out_specs=pl.BlockSpec((_B, _CB), lambda j: (0, j)),
        out_shape=jax.ShapeDtypeStruct((_B, _CP), jnp.float32),
    )(weightsT, z)


def kernel(x, feat_W, feat_b, cls_W, cls_b):
    z = _featurize(x, feat_W, feat_b)
    supports = jnp.concatenate(
        [cls_W, z, jnp.zeros((_NP - _N, _D), jnp.float32)], axis=0)
    clsW_pad = jnp.concatenate(
        [cls_W, jnp.zeros((_CP - _C, _D), jnp.float32)], axis=0)
    clsb_pad = jnp.concatenate(
        [cls_b, jnp.full((_CP - _C,), -1e30, jnp.float32)]).reshape(1, _CP)

    ent, yidx, rn2 = _row_stats(supports, clsW_pad, clsb_pad)
    row_ids = jax.lax.broadcasted_iota(jnp.int32, (_NP, 1), 0)
    yidx = jnp.where(row_ids < _N, yidx, -1)

    coef = _rank_coef(ent, yidx, rn2)
    weightsT = _class_scatter(yidx, coef, supports)
    out = _final_out(weightsT, z)
    return out[:, :_C]


# speedup vs baseline: 1.3054x; 1.3054x over previous
import functools

import jax
import jax.numpy as jnp
from jax import lax
from jax.experimental import pallas as pl
from jax.experimental.pallas import tpu as pltpu

_B = 256
_DIN = 1024
_D = 2048
_C = 1000
_K = 100
_N = _C + _B
_NP = 1280
_CP = 1024
_RB = 256
_CB = 256


def _feat_body(x_ref, w_ref, b_ref, o_ref):
    o_ref[...] = (
        jax.lax.dot_general(
            x_ref[...], w_ref[...], (((1,), (1,)), ((), ())),
            preferred_element_type=jnp.float32)
        + b_ref[...]
    )


def _featurize(x, feat_W, feat_b):
    fb = feat_b.reshape(1, _D)
    return pl.pallas_call(
        _feat_body,
        grid=(4,),
        in_specs=[
            pl.BlockSpec((_B, _DIN), lambda j: (0, 0)),
            pl.BlockSpec((_D // 4, _DIN), lambda j: (j, 0)),
            pl.BlockSpec((1, _D // 4), lambda j: (0, j)),
        ],
        out_specs=pl.BlockSpec((_B, _D // 4), lambda j: (0, j)),
        out_shape=jax.ShapeDtypeStruct((_B, _D), jnp.float32),
    )(x, feat_W, fb)


def _stats_body(sup_ref, w_ref, b_ref, ent_ref, idx_ref, rn2_ref):
    lhs = sup_ref[...]
    logits = jax.lax.dot_general(
        lhs, w_ref[...], (((1,), (1,)), ((), ())),
        preferred_element_type=jnp.float32) + b_ref[...]
    m = jnp.max(logits, axis=1, keepdims=True)
    e = jnp.exp(logits - m)
    s = jnp.sum(e, axis=1, keepdims=True)
    t = jnp.sum(e * logits, axis=1, keepdims=True)
    cols = jax.lax.broadcasted_iota(jnp.int32, logits.shape, 1)
    idx_ref[...] = jnp.min(
        jnp.where(logits == m, cols, jnp.int32(2**30)), axis=1, keepdims=True)
    ent_ref[...] = m + jnp.log(s) - t / s
    rn2_ref[...] = jnp.sum(lhs * lhs, axis=1, keepdims=True)


def _row_stats(supports_pad, cls_W, cls_b):
    return pl.pallas_call(
        _stats_body,
        grid=(_NP // _RB,),
        in_specs=[
            pl.BlockSpec((_RB, _D), lambda i: (i, 0)),
            pl.BlockSpec((_C, _D), lambda i: (0, 0)),
            pl.BlockSpec((1, _C), lambda i: (0, 0)),
        ],
        out_specs=[
            pl.BlockSpec((_RB, 1), lambda i: (i, 0)),
            pl.BlockSpec((_RB, 1), lambda i: (i, 0)),
            pl.BlockSpec((_RB, 1), lambda i: (i, 0)),
        ],
        out_shape=[
            jax.ShapeDtypeStruct((_NP, 1), jnp.float32),
            jax.ShapeDtypeStruct((_NP, 1), jnp.int32),
            jax.ShapeDtypeStruct((_NP, 1), jnp.float32),
        ],
        compiler_params=pltpu.CompilerParams(
            dimension_semantics=("arbitrary",)),
    )(supports_pad, cls_W, cls_b.reshape(1, _C))


def _rank_body(ent_r_ref, y_r_ref, ent_c_ref, y_c_ref, rn2_ref, coef_ref):
    i = pl.program_id(0)
    ent_r = ent_r_ref[...]
    y_r = y_r_ref[...]
    ent_c = ent_c_ref[...]
    y_c = y_c_ref[...]
    idx_r = jax.lax.broadcasted_iota(jnp.int32, (_RB, _NP), 1)
    idx_c = jax.lax.broadcasted_iota(jnp.int32, (_RB, _NP), 0) + i * _RB
    same = y_r == y_c
    earlier = (ent_r < ent_c) | ((ent_r == ent_c) & (idx_r < idx_c))
    rank = jnp.sum((same & earlier).astype(jnp.float32), axis=1,
                   keepdims=True)
    valid = (idx_c[:, :1] < _N) & (rank < _K)
    coef_ref[...] = jnp.where(
        valid, jax.lax.rsqrt(jnp.maximum(rn2_ref[...], 1e-24)), 0.0)


def _rank_coef(ent, yidx, rn2):
    ent_row = ent.reshape(1, _NP)
    y_row = yidx.reshape(1, _NP)
    return pl.pallas_call(
        _rank_body,
        grid=(_NP // _RB,),
        in_specs=[
            pl.BlockSpec((1, _NP), lambda i: (0, 0)),
            pl.BlockSpec((1, _NP), lambda i: (0, 0)),
            pl.BlockSpec((_RB, 1), lambda i: (i, 0)),
            pl.BlockSpec((_RB, 1), lambda i: (i, 0)),
            pl.BlockSpec((_RB, 1), lambda i: (i, 0)),
        ],
        out_specs=pl.BlockSpec((_RB, 1), lambda i: (i, 0)),
        out_shape=jax.ShapeDtypeStruct((_NP, 1), jnp.float32),
    )(ent_row, y_row, ent, yidx, rn2)


def _scatter_body(y_ref, coef_ref, sup_ref, o_ref, acc_sc):
    c = pl.program_id(0)
    i = pl.program_id(1)
    ni = pl.num_programs(1)
    classes = jax.lax.broadcasted_iota(jnp.int32, (_RB, _CB), 1) + c * _CB
    onehot = jnp.where(y_ref[...] == classes, coef_ref[...], 0.0)
    contrib = jax.lax.dot_general(
        onehot, sup_ref[...], (((0,), (0,)), ((), ())),
        preferred_element_type=jnp.float32)

    @pl.when(i == 0)
    def _init():
        acc_sc[...] = contrib

    @pl.when(i > 0)
    def _acc():
        acc_sc[...] += contrib

    @pl.when(i == ni - 1)
    def _final():
        o_ref[...] = acc_sc[...]


def _class_scatter(yidx, coef, supports_pad):
    return pl.pallas_call(
        _scatter_body,
        grid=(_CP // _CB, _NP // _RB),
        in_specs=[
            pl.BlockSpec((_RB, 1), lambda c, i: (i, 0)),
            pl.BlockSpec((_RB, 1), lambda c, i: (i, 0)),
            pl.BlockSpec((_RB, _D), lambda c, i: (i, 0)),
        ],
        out_specs=pl.BlockSpec((_CB, _D), lambda c, i: (c, 0)),
        out_shape=jax.ShapeDtypeStruct((_CP, _D), jnp.float32),
        scratch_shapes=[pltpu.VMEM((_CB, _D), jnp.float32)],
        compiler_params=pltpu.CompilerParams(
            dimension_semantics=("parallel", "arbitrary")),
    )(yidx, coef, supports_pad)


def _out_body(wt_ref, z_ref, o_ref):
    wt = wt_ref[...]
    rn2 = jnp.sum(wt * wt, axis=1, keepdims=True)
    wn = wt * jax.lax.rsqrt(jnp.maximum(rn2, 1e-24))
    o_ref[...] = jax.lax.dot_general(
        z_ref[...], wn, (((1,), (1,)), ((), ())),
        preferred_element_type=jnp.float32)


def _final_out(weightsT, z):
    return pl.pallas_call(
        _out_body,
        grid=(_CP // _CB,),
        in_specs=[
            pl.BlockSpec((_CB, _D), lambda j: (j, 0)),
            pl.BlockSpec((_B, _D), lambda j: (0, 0)),
        ],
        out_specs=pl.BlockSpec((_B, _CB), lambda j: (0, j)),
        out_shape=jax.ShapeDtypeStruct((_B, _CP), jnp.float32),
    )(weightsT, z)


def kernel(x, feat_W, feat_b, cls_W, cls_b):
    z = _featurize(x, feat_W, feat_b)
    supports = jnp.concatenate(
        [cls_W, z, jnp.zeros((_NP - _N, _D), jnp.float32)], axis=0)

    ent, yidx, rn2 = _row_stats(supports, cls_W, cls_b)
    row_ids = jax.lax.broadcasted_iota(jnp.int32, (_NP, 1), 0)
    yidx = jnp.where(row_ids < _N, yidx, -1)

    coef = _rank_coef(ent, yidx, rn2)
    weightsT = _class_scatter(yidx, coef, supports)
    out = _final_out(weightsT, z)
    return out[:, :_C]


# speedup vs baseline: 1.6414x; 1.2575x over previous
import functools

import jax
import jax.numpy as jnp
from jax import lax
from jax.experimental import pallas as pl
from jax.experimental.pallas import tpu as pltpu

_B = 256
_DIN = 1024
_D = 2048
_C = 1000
_K = 100
_N = _C + _B
_NP = 1280
_CP = 1024
_RB = 256
_CB = 256


def _feat_body(x_ref, w_ref, b_ref, o_ref):
    o_ref[...] = (
        jax.lax.dot_general(
            x_ref[...], w_ref[...], (((1,), (1,)), ((), ())),
            preferred_element_type=jnp.float32)
        + b_ref[...]
    )


def _featurize(x, feat_W, feat_b):
    fb = feat_b.reshape(1, _D)
    return pl.pallas_call(
        _feat_body,
        grid=(4,),
        in_specs=[
            pl.BlockSpec((_B, _DIN), lambda j: (0, 0)),
            pl.BlockSpec((_D // 4, _DIN), lambda j: (j, 0)),
            pl.BlockSpec((1, _D // 4), lambda j: (0, j)),
        ],
        out_specs=pl.BlockSpec((_B, _D // 4), lambda j: (0, j)),
        out_shape=jax.ShapeDtypeStruct((_B, _D), jnp.float32),
    )(x, feat_W, fb)


def _stats_body(sup_ref, w_ref, b_ref, ent_ref, idx_ref, rn2_ref):
    lhs = sup_ref[...]
    logits = jax.lax.dot_general(
        lhs, w_ref[...], (((1,), (1,)), ((), ())),
        preferred_element_type=jnp.float32) + b_ref[...]
    m = jnp.max(logits, axis=1, keepdims=True)
    e = jnp.exp(logits - m)
    s = jnp.sum(e, axis=1, keepdims=True)
    t = jnp.sum(e * logits, axis=1, keepdims=True)
    cols = jax.lax.broadcasted_iota(jnp.int32, logits.shape, 1)
    idx_ref[...] = jnp.min(
        jnp.where(logits == m, cols, jnp.int32(2**30)), axis=1, keepdims=True)
    ent_ref[...] = m + jnp.log(s) - t / s
    rn2_ref[...] = jnp.sum(lhs * lhs, axis=1, keepdims=True)


def _row_stats(supports_pad, cls_W, cls_b):
    return pl.pallas_call(
        _stats_body,
        grid=(_NP // _RB,),
        in_specs=[
            pl.BlockSpec((_RB, _D), lambda i: (i, 0)),
            pl.BlockSpec((_C, _D), lambda i: (0, 0)),
            pl.BlockSpec((1, _C), lambda i: (0, 0)),
        ],
        out_specs=[
            pl.BlockSpec((_RB, 1), lambda i: (i, 0)),
            pl.BlockSpec((_RB, 1), lambda i: (i, 0)),
            pl.BlockSpec((_RB, 1), lambda i: (i, 0)),
        ],
        out_shape=[
            jax.ShapeDtypeStruct((_NP, 1), jnp.float32),
            jax.ShapeDtypeStruct((_NP, 1), jnp.int32),
            jax.ShapeDtypeStruct((_NP, 1), jnp.float32),
        ],
        compiler_params=pltpu.CompilerParams(
            dimension_semantics=("arbitrary",)),
    )(supports_pad, cls_W, cls_b.reshape(1, _C))


def _rank_body(ent_r_ref, y_r_ref, ent_c_ref, y_c_ref, rn2_ref, coef_ref):
    i = pl.program_id(0)
    ent_r = ent_r_ref[...]
    y_r = y_r_ref[...]
    ent_c = ent_c_ref[...]
    y_c = y_c_ref[...]
    idx_r = jax.lax.broadcasted_iota(jnp.int32, (_RB, _NP), 1)
    idx_c = jax.lax.broadcasted_iota(jnp.int32, (_RB, _NP), 0) + i * _RB
    same = y_r == y_c
    earlier = (ent_r < ent_c) | ((ent_r == ent_c) & (idx_r < idx_c))
    rank = jnp.sum((same & earlier).astype(jnp.float32), axis=1,
                   keepdims=True)
    valid = (idx_c[:, :1] < _N) & (rank < _K)
    coef_ref[...] = jnp.where(
        valid, jax.lax.rsqrt(jnp.maximum(rn2_ref[...], 1e-24)), 0.0)


def _rank_coef(ent, yidx, rn2):
    ent_row = ent.reshape(1, _NP)
    y_row = yidx.reshape(1, _NP)
    return pl.pallas_call(
        _rank_body,
        grid=(_NP // _RB,),
        in_specs=[
            pl.BlockSpec((1, _NP), lambda i: (0, 0)),
            pl.BlockSpec((1, _NP), lambda i: (0, 0)),
            pl.BlockSpec((_RB, 1), lambda i: (i, 0)),
            pl.BlockSpec((_RB, 1), lambda i: (i, 0)),
            pl.BlockSpec((_RB, 1), lambda i: (i, 0)),
        ],
        out_specs=pl.BlockSpec((_RB, 1), lambda i: (i, 0)),
        out_shape=jax.ShapeDtypeStruct((_NP, 1), jnp.float32),
    )(ent_row, y_row, ent, yidx, rn2)


def _scatter_body(y_ref, coef_ref, sup_ref, o_ref):
    classes = jax.lax.broadcasted_iota(jnp.int32, (_NP, _CP), 1)
    onehot = jnp.where(y_ref[...] == classes, coef_ref[...], 0.0)
    o_ref[...] = jax.lax.dot_general(
        onehot.astype(jnp.bfloat16), sup_ref[...].astype(jnp.bfloat16),
        (((0,), (0,)), ((), ())),
        preferred_element_type=jnp.float32)


def _class_scatter(yidx, coef, supports_pad):
    return pl.pallas_call(
        _scatter_body,
        in_specs=[
            pl.BlockSpec((_NP, 1), lambda: (0, 0)),
            pl.BlockSpec((_NP, 1), lambda: (0, 0)),
            pl.BlockSpec((_NP, _D), lambda: (0, 0)),
        ],
        out_specs=pl.BlockSpec((_CP, _D), lambda: (0, 0)),
        out_shape=jax.ShapeDtypeStruct((_CP, _D), jnp.float32),
    )(yidx, coef, supports_pad)


def _out_body(wt_ref, z_ref, o_ref):
    wt = wt_ref[...]
    rn2 = jnp.sum(wt * wt, axis=1, keepdims=True)
    wn = wt * jax.lax.rsqrt(jnp.maximum(rn2, 1e-24))
    o_ref[...] = jax.lax.dot_general(
        z_ref[...], wn, (((1,), (1,)), ((), ())),
        preferred_element_type=jnp.float32)


def _final_out(weightsT, z):
    return pl.pallas_call(
        _out_body,
        in_specs=[
            pl.BlockSpec((_CP, _D), lambda: (0, 0)),
            pl.BlockSpec((_B, _D), lambda: (0, 0)),
        ],
        out_specs=pl.BlockSpec((_B, _CP), lambda: (0, 0)),
        out_shape=jax.ShapeDtypeStruct((_B, _CP), jnp.float32),
    )(weightsT, z)


def kernel(x, feat_W, feat_b, cls_W, cls_b):
    z = _featurize(x, feat_W, feat_b)
    supports = jnp.concatenate(
        [cls_W, z, jnp.zeros((_NP - _N, _D), jnp.float32)], axis=0)

    ent, yidx, rn2 = _row_stats(supports, cls_W, cls_b)
    row_ids = jax.lax.broadcasted_iota(jnp.int32, (_NP, 1), 0)
    yidx = jnp.where(row_ids < _N, yidx, -1)

    coef = _rank_coef(ent, yidx, rn2)
    weightsT = _class_scatter(yidx, coef, supports)
    out = _final_out(weightsT, z)
    return out[:, :_C]


# speedup vs baseline: 2.0126x; 1.2261x over previous
import jax
import jax.numpy as jnp
from jax.experimental import pallas as pl
from jax.experimental.pallas import tpu as pltpu

_B = 256
_DIN = 1024
_D = 2048
_C = 1000
_K = 100
_N = _C + _B
_NP = 1280
_CP = 1024
_RB = 200


def _feat_body(x_ref, w_ref, b_ref, o_ref):
    o_ref[...] = (
        jax.lax.dot_general(
            x_ref[...], w_ref[...], (((1,), (1,)), ((), ())),
            preferred_element_type=jnp.float32)
        + b_ref[...]
    )


def _featurize(x, feat_W, feat_b):
    return pl.pallas_call(
        _feat_body,
        grid=(4,),
        in_specs=[
            pl.BlockSpec((_B, _DIN), lambda j: (0, 0)),
            pl.BlockSpec((_D // 4, _DIN), lambda j: (j, 0)),
            pl.BlockSpec((1, _D // 4), lambda j: (0, j)),
        ],
        out_specs=pl.BlockSpec((_B, _D // 4), lambda j: (0, j)),
        out_shape=jax.ShapeDtypeStruct((_B, _D), jnp.float32),
    )(x, feat_W, fb := feat_b.reshape(1, _D))


def _row_block_stats(lhs, w, b):
    logits = jax.lax.dot_general(
        lhs, w, (((1,), (1,)), ((), ())),
        preferred_element_type=jnp.float32) + b
    m = jnp.max(logits, axis=1, keepdims=True)
    e = jnp.exp(logits - m)
    s = jnp.sum(e, axis=1, keepdims=True)
    t = jnp.sum(e * logits, axis=1, keepdims=True)
    cols = jax.lax.broadcasted_iota(jnp.int32, logits.shape, 1)
    bi = jnp.min(jnp.where(logits == m, cols, jnp.int32(2**30)),
                 axis=1, keepdims=True)
    ent = m + jnp.log(s) - t / s
    rn2 = jnp.sum(lhs * lhs, axis=1, keepdims=True)
    return ent, bi, rn2


def _stats_body(w_ref, z_ref, b_ref, ent_ref, idx_ref, rn2_ref):
    s = pl.program_id(0)
    w = w_ref[...]
    b = b_ref[...]

    @pl.when(s < 5)
    def _warmup_rows():
        lhs = w_ref[pl.ds(s * _RB, _RB), :]
        ent, bi, rn2 = _row_block_stats(lhs, w, b)
        ent_ref[pl.ds(s * _RB, _RB), :] = ent
        idx_ref[pl.ds(s * _RB, _RB), :] = bi
        rn2_ref[pl.ds(s * _RB, _RB), :] = rn2

    @pl.when(s == 5)
    def _batch_rows():
        ent, bi, rn2 = _row_block_stats(z_ref[...], w, b)
        ent_ref[pl.ds(_C, _B), :] = ent
        idx_ref[pl.ds(_C, _B), :] = bi
        rn2_ref[pl.ds(_C, _B), :] = rn2
        idx_ref[pl.ds(_N, _NP - _N), :] = jnp.full(
            (_NP - _N, 1), -1, jnp.int32)


def _row_stats(cls_W, z, cls_b):
    return pl.pallas_call(
        _stats_body,
        grid=(6,),
        in_specs=[
            pl.BlockSpec((_C, _D), lambda s: (0, 0)),
            pl.BlockSpec((_B, _D), lambda s: (0, 0)),
            pl.BlockSpec((1, _C), lambda s: (0, 0)),
        ],
        out_specs=[
            pl.BlockSpec((_NP, 1), lambda s: (0, 0)),
            pl.BlockSpec((_NP, 1), lambda s: (0, 0)),
            pl.BlockSpec((_NP, 1), lambda s: (0, 0)),
        ],
        out_shape=[
            jax.ShapeDtypeStruct((_NP, 1), jnp.float32),
            jax.ShapeDtypeStruct((_NP, 1), jnp.int32),
            jax.ShapeDtypeStruct((_NP, 1), jnp.float32),
        ],
        compiler_params=pltpu.CompilerParams(
            dimension_semantics=("arbitrary",)),
    )(cls_W, z, cls_b.reshape(1, _C))


def _rank_body(ent_r_ref, y_r_ref, ent_c_ref, y_c_ref, rn2_ref, coef_ref):
    i = pl.program_id(0)
    ent_r = ent_r_ref[...]
    y_r = y_r_ref[...]
    ent_c = ent_c_ref[...]
    y_c = y_c_ref[...]
    idx_r = jax.lax.broadcasted_iota(jnp.int32, (256, _NP), 1)
    idx_c = jax.lax.broadcasted_iota(jnp.int32, (256, _NP), 0) + i * 256
    same = y_r == y_c
    earlier = (ent_r < ent_c) | ((ent_r == ent_c) & (idx_r < idx_c))
    rank = jnp.sum((same & earlier).astype(jnp.float32), axis=1,
                   keepdims=True)
    valid = (idx_c[:, :1] < _N) & (rank < _K)
    coef_ref[...] = jnp.where(
        valid, jax.lax.rsqrt(jnp.maximum(rn2_ref[...], 1e-24)), 0.0)


def _rank_coef(ent, yidx, rn2):
    ent_row = ent.reshape(1, _NP)
    y_row = yidx.reshape(1, _NP)
    return pl.pallas_call(
        _rank_body,
        grid=(_NP // 256,),
        in_specs=[
            pl.BlockSpec((1, _NP), lambda i: (0, 0)),
            pl.BlockSpec((1, _NP), lambda i: (0, 0)),
            pl.BlockSpec((256, 1), lambda i: (i, 0)),
            pl.BlockSpec((256, 1), lambda i: (i, 0)),
            pl.BlockSpec((256, 1), lambda i: (i, 0)),
        ],
        out_specs=pl.BlockSpec((256, 1), lambda i: (i, 0)),
        out_shape=jax.ShapeDtypeStruct((_NP, 1), jnp.float32),
    )(ent_row, y_row, ent, yidx, rn2)


def _out_body(y_ref, coef_ref, w_ref, z_ref, o_ref):
    yA = y_ref[pl.ds(0, _C), :]
    cA = coef_ref[pl.ds(0, _C), :]
    yB = y_ref[pl.ds(_C, _B), :]
    cB = coef_ref[pl.ds(_C, _B), :]
    clsA = jax.lax.broadcasted_iota(jnp.int32, (_C, _CP), 1)
    clsB = jax.lax.broadcasted_iota(jnp.int32, (_B, _CP), 1)
    ohA = jnp.where(yA == clsA, cA, 0.0).astype(jnp.bfloat16)
    ohB = jnp.where(yB == clsB, cB, 0.0).astype(jnp.bfloat16)
    z = z_ref[...]
    wT = jax.lax.dot_general(
        ohA, w_ref[...].astype(jnp.bfloat16), (((0,), (0,)), ((), ())),
        preferred_element_type=jnp.float32)
    wT = wT + jax.lax.dot_general(
        ohB, z.astype(jnp.bfloat16), (((0,), (0,)), ((), ())),
        preferred_element_type=jnp.float32)
    rn2 = jnp.sum(wT * wT, axis=1, keepdims=True)
    wn = wT * jax.lax.rsqrt(jnp.maximum(rn2, 1e-24))
    res = jax.lax.dot_general(
        z, wn, (((1,), (1,)), ((), ())),
        preferred_element_type=jnp.float32)
    o_ref[...] = res[:, :_C]


def _final_out(yidx, coef, cls_W, z):
    return pl.pallas_call(
        _out_body,
        in_specs=[
            pl.BlockSpec((_NP, 1), lambda: (0, 0)),
            pl.BlockSpec((_NP, 1), lambda: (0, 0)),
            pl.BlockSpec((_C, _D), lambda: (0, 0)),
            pl.BlockSpec((_B, _D), lambda: (0, 0)),
        ],
        out_specs=pl.BlockSpec((_B, _C), lambda: (0, 0)),
        out_shape=jax.ShapeDtypeStruct((_B, _C), jnp.float32),
    )(yidx, coef, cls_W, z)


def kernel(x, feat_W, feat_b, cls_W, cls_b):
    z = _featurize(x, feat_W, feat_b)
    ent, yidx, rn2 = _row_stats(cls_W, z, cls_b)
    coef = _rank_coef(ent, yidx, rn2)
    return _final_out(yidx, coef, cls_W, z)


# speedup vs baseline: 2.3939x; 1.1895x over previous
import jax
import jax.numpy as jnp
from jax.experimental import pallas as pl
from jax.experimental.pallas import tpu as pltpu

_B = 256
_DIN = 1024
_D = 2048
_C = 1000
_K = 100
_N = _C + _B
_NP = 1280
_CP = 1024
_RB = 200
_ZC = 512


def _row_block_stats(lhs, w, b):
    logits = jax.lax.dot_general(
        lhs, w, (((1,), (1,)), ((), ())),
        preferred_element_type=jnp.float32) + b
    m = jnp.max(logits, axis=1, keepdims=True)
    e = jnp.exp(logits - m)
    s = jnp.sum(e, axis=1, keepdims=True)
    t = jnp.sum(e * logits, axis=1, keepdims=True)
    cols = jax.lax.broadcasted_iota(jnp.int32, logits.shape, 1)
    bi = jnp.min(jnp.where(logits == m, cols, jnp.int32(2**30)),
                 axis=1, keepdims=True)
    ent = m + jnp.log(s) - t / s
    rn2 = jnp.sum(lhs * lhs, axis=1, keepdims=True)
    return ent, bi, rn2


def _k12_body(w_ref, x_ref, fw_ref, fb_ref, b_ref,
              z_ref, ent_ref, idx_ref, rn2_ref, z_sc):
    s = pl.program_id(0)
    w = w_ref[...]
    b = b_ref[...]

    @pl.when(s <= 3)
    def _z_chunk():
        zc = jax.lax.dot_general(
            x_ref[...], fw_ref[...], (((1,), (1,)), ((), ())),
            preferred_element_type=jnp.float32) + fb_ref[...]
        z_sc[:, pl.ds(s * _ZC, _ZC)] = zc
        z_ref[...] = zc

    @pl.when(s < 5)
    def _warmup_rows():
        lhs = w_ref[pl.ds(s * _RB, _RB), :]
        ent, bi, rn2 = _row_block_stats(lhs, w, b)
        ent_ref[pl.ds(s * _RB, _RB), :] = ent
        idx_ref[pl.ds(s * _RB, _RB), :] = bi
        rn2_ref[pl.ds(s * _RB, _RB), :] = rn2

    @pl.when(s == 5)
    def _batch_rows():
        ent, bi, rn2 = _row_block_stats(z_sc[...], w, b)
        ent_ref[pl.ds(_C, _B), :] = ent
        idx_ref[pl.ds(_C, _B), :] = bi
        rn2_ref[pl.ds(_C, _B), :] = rn2
        idx_ref[pl.ds(_N, _NP - _N), :] = jnp.full(
            (_NP - _N, 1), -1, jnp.int32)


def _k12(x, feat_W, feat_b, cls_W, cls_b):
    return pl.pallas_call(
        _k12_body,
        grid=(6,),
        in_specs=[
            pl.BlockSpec((_C, _D), lambda s: (0, 0)),
            pl.BlockSpec((_B, _DIN), lambda s: (0, 0)),
            pl.BlockSpec((_ZC, _DIN), lambda s: (jnp.minimum(s, 3), 0)),
            pl.BlockSpec((1, _ZC), lambda s: (0, jnp.minimum(s, 3))),
            pl.BlockSpec((1, _C), lambda s: (0, 0)),
        ],
        out_specs=[
            pl.BlockSpec((_B, _ZC), lambda s: (0, jnp.minimum(s, 3))),
            pl.BlockSpec((_NP, 1), lambda s: (0, 0)),
            pl.BlockSpec((_NP, 1), lambda s: (0, 0)),
            pl.BlockSpec((_NP, 1), lambda s: (0, 0)),
        ],
        out_shape=[
            jax.ShapeDtypeStruct((_B, _D), jnp.float32),
            jax.ShapeDtypeStruct((_NP, 1), jnp.float32),
            jax.ShapeDtypeStruct((_NP, 1), jnp.int32),
            jax.ShapeDtypeStruct((_NP, 1), jnp.float32),
        ],
        scratch_shapes=[pltpu.VMEM((_B, _D), jnp.float32)],
        compiler_params=pltpu.CompilerParams(
            dimension_semantics=("arbitrary",)),
    )(cls_W, x, feat_W, feat_b.reshape(1, _D), cls_b.reshape(1, _C))


def _k345_body(ent_r_ref, y_r_ref, ent_c_ref, y_c_ref, rn2_ref,
               w_ref, z_ref, o_ref):
    ent_r = ent_r_ref[...]
    y_r = y_r_ref[...]
    ent_c = ent_c_ref[...]
    y_c = y_c_ref[...]
    idx_r = jax.lax.broadcasted_iota(jnp.int32, (_NP, _NP), 1)
    idx_c = jax.lax.broadcasted_iota(jnp.int32, (_NP, _NP), 0)
    same = y_r == y_c
    earlier = (ent_r < ent_c) | ((ent_r == ent_c) & (idx_r < idx_c))
    rank = jnp.sum((same & earlier).astype(jnp.float32), axis=1,
                   keepdims=True)
    valid = (idx_c[:, :1] < _N) & (rank < _K)
    coef = jnp.where(
        valid, jax.lax.rsqrt(jnp.maximum(rn2_ref[...], 1e-24)), 0.0)

    yA = y_c[:_C, :]
    cA = coef[:_C, :]
    yB = y_c[_C:_N, :]
    cB = coef[_C:_N, :]
    clsA = jax.lax.broadcasted_iota(jnp.int32, (_C, _CP), 1)
    clsB = jax.lax.broadcasted_iota(jnp.int32, (_B, _CP), 1)
    ohA = jnp.where(yA == clsA, cA, 0.0).astype(jnp.bfloat16)
    ohB = jnp.where(yB == clsB, cB, 0.0).astype(jnp.bfloat16)
    z = z_ref[...]
    wT = jax.lax.dot_general(
        ohA, w_ref[...].astype(jnp.bfloat16), (((0,), (0,)), ((), ())),
        preferred_element_type=jnp.float32)
    wT = wT + jax.lax.dot_general(
        ohB, z.astype(jnp.bfloat16), (((0,), (0,)), ((), ())),
        preferred_element_type=jnp.float32)
    wn2 = jnp.sum(wT * wT, axis=1, keepdims=True)
    wn = wT * jax.lax.rsqrt(jnp.maximum(wn2, 1e-24))
    res = jax.lax.dot_general(
        z, wn, (((1,), (1,)), ((), ())),
        preferred_element_type=jnp.float32)
    o_ref[...] = res[:, :_C]


def _k345(ent, yidx, rn2, cls_W, z):
    return pl.pallas_call(
        _k345_body,
        in_specs=[
            pl.BlockSpec((1, _NP), lambda: (0, 0)),
            pl.BlockSpec((1, _NP), lambda: (0, 0)),
            pl.BlockSpec((_NP, 1), lambda: (0, 0)),
            pl.BlockSpec((_NP, 1), lambda: (0, 0)),
            pl.BlockSpec((_NP, 1), lambda: (0, 0)),
            pl.BlockSpec((_C, _D), lambda: (0, 0)),
            pl.BlockSpec((_B, _D), lambda: (0, 0)),
        ],
        out_specs=pl.BlockSpec((_B, _C), lambda: (0, 0)),
        out_shape=jax.ShapeDtypeStruct((_B, _C), jnp.float32),
    )(ent.reshape(1, _NP), yidx.reshape(1, _NP), ent, yidx, rn2, cls_W, z)


def kernel(x, feat_W, feat_b, cls_W, cls_b):
    z, ent, yidx, rn2 = _k12(x, feat_W, feat_b, cls_W, cls_b)
    return _k345(ent, yidx, rn2, cls_W, z)


# speedup vs baseline: 3.3711x; 1.4082x over previous
import jax
import jax.numpy as jnp
from jax import lax
from jax.experimental import pallas as pl
from jax.experimental.pallas import tpu as pltpu

_B = 256
_DIN = 1024
_D = 2048
_C = 1000
_K = 100
_N = _C + _B
_NP = 1280
_CP = 1024
_RB = 200
_ZC = 512


def _row_block_stats(lhs, w, b):
    logits = jax.lax.dot_general(
        lhs, w, (((1,), (1,)), ((), ())),
        preferred_element_type=jnp.float32) + b
    m = jnp.max(logits, axis=1, keepdims=True)
    e = jnp.exp(logits - m)
    s = jnp.sum(e, axis=1, keepdims=True)
    t = jnp.sum(e * logits, axis=1, keepdims=True)
    cols = jax.lax.broadcasted_iota(jnp.int32, logits.shape, 1)
    bi = jnp.min(jnp.where(logits == m, cols, jnp.int32(2**30)),
                 axis=1, keepdims=True)
    ent = m + jnp.log(s) - t / s
    rn2 = jnp.sum(lhs * lhs, axis=1, keepdims=True)
    invn = jax.lax.rsqrt(jnp.maximum(rn2, 1e-24))
    return ent, bi.astype(jnp.float32), invn


def _body(w_ref, x_ref, fw_ref, fb_ref, b_ref, o_ref,
          z_sc, ent_sc, yf_sc, inv_sc):
    s = pl.program_id(0)

    @pl.when(s <= 3)
    def _z_chunk():
        zc = jax.lax.dot_general(
            x_ref[...], fw_ref[...], (((1,), (1,)), ((), ())),
            preferred_element_type=jnp.float32) + fb_ref[...]
        z_sc[:, pl.ds(s * _ZC, _ZC)] = zc

    @pl.when(s < 5)
    def _warmup_rows():
        lhs = w_ref[pl.ds(s * _RB, _RB), :]
        ent, yf, invn = _row_block_stats(lhs, w_ref[...], b_ref[...])
        ent_sc[pl.ds(s * _RB, _RB), :] = ent
        yf_sc[pl.ds(s * _RB, _RB), :] = yf
        inv_sc[pl.ds(s * _RB, _RB), :] = invn

    @pl.when(s == 5)
    def _batch_rows():
        ent, yf, invn = _row_block_stats(z_sc[...], w_ref[...], b_ref[...])
        ent_sc[pl.ds(_C, _B), :] = ent
        yf_sc[pl.ds(_C, _B), :] = yf
        inv_sc[pl.ds(_C, _B), :] = invn
        yf_sc[pl.ds(_N, _NP - _N), :] = jnp.full(
            (_NP - _N, 1), -1.0, jnp.float32)

    @pl.when(s == 6)
    def _filter_and_out():
        ent_c = ent_sc[...]
        yf_c = yf_sc[...]
        idx_r = jax.lax.broadcasted_iota(jnp.int32, (_NP, _NP), 1)
        idx_c = jax.lax.broadcasted_iota(jnp.int32, (_NP, _NP), 0)
        eye = (idx_r == idx_c).astype(jnp.float32)
        cat = jnp.concatenate([ent_c, yf_c], axis=1)
        rows = jax.lax.dot_general(
            cat, eye, (((0,), (0,)), ((), ())),
            preferred_element_type=jnp.float32)
        ent_r = rows[0:1, :]
        yf_r = rows[1:2, :]
        same = yf_r == yf_c
        earlier = ((ent_r < ent_c) | ((ent_r == ent_c) & (idx_r < idx_c))
                   ) & (idx_r != idx_c)
        rank = jnp.sum((same & earlier).astype(jnp.float32), axis=1,
                       keepdims=True)
        valid = (idx_c[:, :1] < _N) & (rank < _K)
        coef = jnp.where(valid, inv_sc[...], 0.0)

        y_i32 = yf_c.astype(jnp.int32)
        yA = y_i32[:_C, :]
        cA = coef[:_C, :]
        yB = y_i32[_C:_N, :]
        cB = coef[_C:_N, :]
        clsA = jax.lax.broadcasted_iota(jnp.int32, (_C, _CP), 1)
        clsB = jax.lax.broadcasted_iota(jnp.int32, (_B, _CP), 1)
        ohA = jnp.where(yA == clsA, cA, 0.0).astype(jnp.bfloat16)
        ohB = jnp.where(yB == clsB, cB, 0.0).astype(jnp.bfloat16)
        z = z_sc[...]
        wT = jax.lax.dot_general(
            ohA, w_ref[...].astype(jnp.bfloat16), (((0,), (0,)), ((), ())),
            preferred_element_type=jnp.float32)
        wT = wT + jax.lax.dot_general(
            ohB, z.astype(jnp.bfloat16), (((0,), (0,)), ((), ())),
            preferred_element_type=jnp.float32)
        wn2 = jnp.sum(wT * wT, axis=1, keepdims=True)
        wn = wT * jax.lax.rsqrt(jnp.maximum(wn2, 1e-24))
        res = jax.lax.dot_general(
            z, wn, (((1,), (1,)), ((), ())),
            preferred_element_type=jnp.float32)
        o_ref[...] = res[:, :_C]


def kernel(x, feat_W, feat_b, cls_W, cls_b):
    return pl.pallas_call(
        _body,
        grid=(7,),
        in_specs=[
            pl.BlockSpec((_C, _D), lambda s: (0, 0)),
            pl.BlockSpec((_B, _DIN), lambda s: (0, 0)),
            pl.BlockSpec((_ZC, _DIN), lambda s: (jnp.minimum(s, 3), 0)),
            pl.BlockSpec((1, _ZC), lambda s: (0, jnp.minimum(s, 3))),
            pl.BlockSpec((1, _C), lambda s: (0, 0)),
        ],
        out_specs=pl.BlockSpec((_B, _C), lambda s: (0, 0)),
        out_shape=jax.ShapeDtypeStruct((_B, _C), jnp.float32),
        scratch_shapes=[
            pltpu.VMEM((_B, _D), jnp.float32),
            pltpu.VMEM((_NP, 1), jnp.float32),
            pltpu.VMEM((_NP, 1), jnp.float32),
            pltpu.VMEM((_NP, 1), jnp.float32),
        ],
        compiler_params=pltpu.CompilerParams(
            dimension_semantics=("arbitrary",)),
    )(cls_W, x, feat_W, feat_b.reshape(1, _D), cls_b.reshape(1, _C))


# speedup vs baseline: 3.3764x; 1.0016x over previous
import jax
import jax.numpy as jnp
from jax import lax
from jax.experimental import pallas as pl
from jax.experimental.pallas import tpu as pltpu

_B = 256
_DIN = 1024
_D = 2048
_C = 1000
_K = 100
_N = _C + _B
_NP = 1280
_CP = 1024
_RB = 200
_ZC = 512


def _row_block_stats(lhs, w, b):
    logits = jax.lax.dot_general(
        lhs, w, (((1,), (1,)), ((), ())),
        preferred_element_type=jnp.float32) + b
    m = jnp.max(logits, axis=1, keepdims=True)
    e = jnp.exp(logits - m)
    s = jnp.sum(e, axis=1, keepdims=True)
    t = jnp.sum(e * logits, axis=1, keepdims=True)
    cols = jax.lax.broadcasted_iota(jnp.int32, logits.shape, 1)
    bi = jnp.min(jnp.where(logits == m, cols, jnp.int32(2**30)),
                 axis=1, keepdims=True)
    ent = m + jnp.log(s) - t / s
    rn2 = jnp.sum(lhs * lhs, axis=1, keepdims=True)
    invn = jax.lax.rsqrt(jnp.maximum(rn2, 1e-24))
    return ent, bi.astype(jnp.float32), invn


def _body(w_ref, x_ref, fw_ref, fb_ref, b_ref, o_ref,
          z_sc, ent_sc, yf_sc, inv_sc):
    s = pl.program_id(0)

    @pl.when(s <= 3)
    def _z_chunk():
        zc = jax.lax.dot_general(
            x_ref[...], fw_ref[...], (((1,), (1,)), ((), ())),
            preferred_element_type=jnp.float32) + fb_ref[...]
        z_sc[:, pl.ds(s * _ZC, _ZC)] = zc

    @pl.when(s < 5)
    def _warmup_rows():
        lhs = w_ref[pl.ds(s * _RB, _RB), :]
        ent, yf, invn = _row_block_stats(lhs, w_ref[...], b_ref[...])
        ent_sc[pl.ds(s * _RB, _RB), :] = ent
        yf_sc[pl.ds(s * _RB, _RB), :] = yf
        inv_sc[pl.ds(s * _RB, _RB), :] = invn

    @pl.when(s == 5)
    def _batch_rows():
        ent, yf, invn = _row_block_stats(z_sc[...], w_ref[...], b_ref[...])
        ent_sc[pl.ds(_C, _B), :] = ent
        yf_sc[pl.ds(_C, _B), :] = yf
        inv_sc[pl.ds(_C, _B), :] = invn
        yf_sc[pl.ds(_N, _NP - _N), :] = jnp.full(
            (_NP - _N, 1), -1.0, jnp.float32)

    @pl.when(s == 6)
    def _filter_and_out():
        ent_c = ent_sc[...]
        yf_c = yf_sc[...]
        idx_r = jax.lax.broadcasted_iota(jnp.int32, (_NP, _NP), 1)
        idx_c = jax.lax.broadcasted_iota(jnp.int32, (_NP, _NP), 0)
        cat = jnp.concatenate([ent_c, yf_c], axis=1)
        rows = jnp.transpose(cat)
        ent_r = rows[0:1, :]
        yf_r = rows[1:2, :]
        same = yf_r == yf_c
        earlier = (ent_r < ent_c) | ((ent_r == ent_c) & (idx_r < idx_c))
        rank = jnp.sum(same & earlier, axis=1,
                       keepdims=True)
        valid = (idx_c[:, :1] < _N) & (rank < _K)
        coef = jnp.where(valid, inv_sc[...], 0.0)

        y_i32 = yf_c.astype(jnp.int32)
        yA = y_i32[:_C, :]
        cA = coef[:_C, :]
        yB = y_i32[_C:_N, :]
        cB = coef[_C:_N, :]
        clsA = jax.lax.broadcasted_iota(jnp.int32, (_C, _CP), 1)
        clsB = jax.lax.broadcasted_iota(jnp.int32, (_B, _CP), 1)
        ohA = jnp.where(yA == clsA, cA, 0.0).astype(jnp.bfloat16)
        ohB = jnp.where(yB == clsB, cB, 0.0).astype(jnp.bfloat16)
        z = z_sc[...]
        wT = jax.lax.dot_general(
            ohA, w_ref[...].astype(jnp.bfloat16), (((0,), (0,)), ((), ())),
            preferred_element_type=jnp.float32)
        wT = wT + jax.lax.dot_general(
            ohB, z.astype(jnp.bfloat16), (((0,), (0,)), ((), ())),
            preferred_element_type=jnp.float32)
        wn2 = jnp.sum(wT * wT, axis=1, keepdims=True)
        wn = wT * jax.lax.rsqrt(jnp.maximum(wn2, 1e-24))
        res = jax.lax.dot_general(
            z, wn, (((1,), (1,)), ((), ())),
            preferred_element_type=jnp.float32)
        o_ref[...] = res[:, :_C]


def kernel(x, feat_W, feat_b, cls_W, cls_b):
    return pl.pallas_call(
        _body,
        grid=(7,),
        in_specs=[
            pl.BlockSpec((_C, _D), lambda s: (0, 0)),
            pl.BlockSpec((_B, _DIN), lambda s: (0, 0)),
            pl.BlockSpec((_ZC, _DIN), lambda s: (jnp.minimum(s, 3), 0)),
            pl.BlockSpec((1, _ZC), lambda s: (0, jnp.minimum(s, 3))),
            pl.BlockSpec((1, _C), lambda s: (0, 0)),
        ],
        out_specs=pl.BlockSpec((_B, _C), lambda s: (0, 0)),
        out_shape=jax.ShapeDtypeStruct((_B, _C), jnp.float32),
        scratch_shapes=[
            pltpu.VMEM((_B, _D), jnp.float32),
            pltpu.VMEM((_NP, 1), jnp.float32),
            pltpu.VMEM((_NP, 1), jnp.float32),
            pltpu.VMEM((_NP, 1), jnp.float32),
        ],
        compiler_params=pltpu.CompilerParams(
            dimension_semantics=("arbitrary",)),
    )(cls_W, x, feat_W, feat_b.reshape(1, _D), cls_b.reshape(1, _C))
